# baseline (speedup 1.0000x reference)
import functools

import jax
import jax.numpy as jnp
from jax import lax
from jax.experimental import pallas as pl
from jax.experimental.pallas import tpu as pltpu
from jax.experimental.pallas import tpu_sc as plsc

N = 10000
E = 320000
F = 128
OUT = 60

NCORES = 2
NSUB = 16
NW = NCORES * NSUB
NPAD = 10240
ROWS_PER_TILE = NPAD // NSUB
CHUNK = 128
CPROC = 80
BLK = 8
CIDX = CPROC + 2 * BLK
EPAD = NW * CPROC * CHUNK
DEGW = 128

@functools.cache
def _deg_kernel():
  mesh = plsc.VectorSubcoreMesh(core_axis_name="c", subcore_axis_name="s",
                                num_cores=NCORES, num_subcores=NSUB)

  @functools.partial(
      pl.kernel,
      out_type=jax.ShapeDtypeStruct((NCORES, NPAD, DEGW), jnp.float32),
      mesh=mesh,
      scratch_types=[
          pltpu.VMEM((CIDX, CHUNK), jnp.int32),
          pltpu.VMEM((CHUNK, DEGW), jnp.float32),
          pltpu.VMEM((8, DEGW), jnp.float32),
          pltpu.VMEM_SHARED((NPAD, DEGW), jnp.float32),
      ],
  )
  def deg_kernel(ones_hbm, dst_hbm, out_hbm, dst_v, ones_v, zb_v, acc_sh):
    c = lax.axis_index("c")
    s = lax.axis_index("s")
    wid = c * NSUB + s
    for i in range(8):
        for j in range(DEGW // 16):
            zb_v[i, pl.ds(j * 16, 16)] = jnp.zeros((16,), jnp.float32)
    pltpu.sync_copy(ones_hbm, ones_v)
    row0 = s * ROWS_PER_TILE
    for k in range(ROWS_PER_TILE // 8):
        pltpu.sync_copy(zb_v, acc_sh.at[pl.ds(row0 + k * 8, 8), :])
    plsc.subcore_barrier()
    pltpu.sync_copy(dst_hbm.at[wid], dst_v)

    def body(ci, carry):
        pltpu.sync_copy(ones_v, acc_sh.at[dst_v.at[ci]], add=True)
        return carry

    lax.fori_loop(0, CPROC, body, 0)
    plsc.subcore_barrier()
    pltpu.sync_copy(acc_sh.at[pl.ds(row0, ROWS_PER_TILE), :],
                    out_hbm.at[c, pl.ds(row0, ROWS_PER_TILE), :])

  return deg_kernel


@functools.cache
def _agg_kernel():
  mesh = plsc.VectorSubcoreMesh(core_axis_name="c", subcore_axis_name="s",
                                num_cores=NCORES, num_subcores=NSUB)

  @functools.partial(
      pl.kernel,
      out_type=jax.ShapeDtypeStruct((NCORES, NPAD, F), jnp.float32),
      mesh=mesh,
      scratch_types=[
          pltpu.VMEM((CPROC, CHUNK), jnp.int32),
          pltpu.VMEM((BLK, CHUNK), jnp.int32),
          pltpu.VMEM((BLK, CHUNK), jnp.int32),
          pltpu.VMEM((CHUNK, F), jnp.float32),
          pltpu.VMEM((CHUNK, F), jnp.float32),
          pltpu.VMEM((8, F), jnp.float32),
          pltpu.VMEM_SHARED((NPAD, F), jnp.float32),
          pltpu.SemaphoreType.DMA,
          pltpu.SemaphoreType.DMA,
          pltpu.SemaphoreType.DMA,
      ],
  )
  def agg_kernel(g_hbm, src_hbm, dst_hbm, out_hbm,
                 dst_v, sa_v, sb_v, ra_v, rb_v, zb_v, acc_sh,
                 sema, semb, semi):
    c = lax.axis_index("c")
    s = lax.axis_index("s")
    wid = c * NSUB + s
    for i in range(8):
        for j in range(F // 16):
            zb_v[i, pl.ds(j * 16, 16)] = jnp.zeros((16,), jnp.float32)
    row0 = s * ROWS_PER_TILE
    for k in range(ROWS_PER_TILE // 8):
        pltpu.sync_copy(zb_v, acc_sh.at[pl.ds(row0 + k * 8, 8), :])
    plsc.subcore_barrier()

    pltpu.sync_copy(dst_hbm.at[wid, pl.ds(0, CPROC)], dst_v)
    pltpu.sync_copy(src_hbm.at[wid, pl.ds(0, BLK)], sa_v)
    pltpu.async_copy(g_hbm.at[sa_v.at[0]], ra_v, sema)
    pltpu.async_copy(src_hbm.at[wid, pl.ds(BLK, BLK)], sb_v, semi)

    def body(p, carry):
        c0 = 2 * BLK * p
        for j in range(2 * BLK):
            cur_r, cur_sem = (ra_v, sema) if j % 2 == 0 else (rb_v, semb)
            nxt_r, nxt_sem = (rb_v, semb) if j % 2 == 0 else (ra_v, sema)
            cur_idx = sa_v.at[j] if j < BLK else sb_v.at[j - BLK]
            pltpu.make_async_copy(g_hbm.at[cur_idx], cur_r,
                                  cur_sem).wait()
            if j == BLK - 1:
                pltpu.make_async_copy(src_hbm.at[wid, pl.ds(0, BLK)],
                                      sb_v, semi).wait()
                pltpu.async_copy(g_hbm.at[sb_v.at[0]], nxt_r, nxt_sem)
                pltpu.sync_copy(cur_r, acc_sh.at[dst_v.at[c0 + j]], add=True)
                pltpu.async_copy(
                    src_hbm.at[wid, pl.ds(c0 + 2 * BLK, BLK)], sa_v, semi)
            elif j == 2 * BLK - 1:
                pltpu.make_async_copy(src_hbm.at[wid, pl.ds(0, BLK)],
                                      sa_v, semi).wait()
                pltpu.async_copy(g_hbm.at[sa_v.at[0]], nxt_r, nxt_sem)
                pltpu.sync_copy(cur_r, acc_sh.at[dst_v.at[c0 + j]], add=True)
                pltpu.async_copy(
                    src_hbm.at[wid, pl.ds(c0 + 3 * BLK, BLK)], sb_v, semi)
            else:
                idx = sa_v.at[j + 1] if j < BLK else sb_v.at[j - BLK + 1]
                pltpu.async_copy(g_hbm.at[idx], nxt_r, nxt_sem)
                pltpu.sync_copy(cur_r, acc_sh.at[dst_v.at[c0 + j]], add=True)
        return carry

    lax.fori_loop(0, CPROC // (2 * BLK), body, 0)
    pltpu.make_async_copy(g_hbm.at[sa_v.at[0]], ra_v, sema).wait()
    pltpu.make_async_copy(src_hbm.at[wid, pl.ds(0, BLK)], sb_v, semi).wait()
    plsc.subcore_barrier()
    pltpu.sync_copy(acc_sh.at[pl.ds(row0, ROWS_PER_TILE), :],
                    out_hbm.at[c, pl.ds(row0, ROWS_PER_TILE), :])

  return agg_kernel


BM = 1024
GRID = NPAD // BM


def _tc1_body(x_ref, w1_ref, deg_ref, g1_ref, dinv_ref):
    deg = deg_ref[0, :, :] + deg_ref[1, :, :]
    dinv = lax.rsqrt(deg[:, 0:1] + 1.0)
    h = jnp.dot(x_ref[...], w1_ref[...], preferred_element_type=jnp.float32)
    g1_ref[...] = h * dinv
    dinv_ref[...] = jnp.broadcast_to(dinv, (BM, F))


def _tc1(xp, W1, degP):
    return pl.pallas_call(
        _tc1_body,
        grid=(GRID,),
        in_specs=[
            pl.BlockSpec((BM, F), lambda i: (i, 0)),
            pl.BlockSpec((F, F), lambda i: (0, 0)),
            pl.BlockSpec((NCORES, BM, DEGW), lambda i: (0, i, 0)),
        ],
        out_specs=[
            pl.BlockSpec((BM, F), lambda i: (i, 0)),
            pl.BlockSpec((BM, F), lambda i: (i, 0)),
        ],
        out_shape=[
            jax.ShapeDtypeStruct((NPAD, F), jnp.float32),
            jax.ShapeDtypeStruct((NPAD, F), jnp.float32),
        ],
    )(xp, W1, degP)


def _tc2_body(p_ref, g1_ref, dinv_ref, b1_ref, w2_ref, g2_ref):
    dinv = dinv_ref[...]
    t = jnp.tanh((p_ref[0] + p_ref[1] + g1_ref[...]) * dinv + b1_ref[...])
    g2_ref[...] = jnp.dot(t, w2_ref[...],
                          preferred_element_type=jnp.float32) * dinv


def _tc2(P, g1, dinv_b, b1, W2):
    return pl.pallas_call(
        _tc2_body,
        grid=(GRID,),
        in_specs=[
            pl.BlockSpec((NCORES, BM, F), lambda i: (0, i, 0)),
            pl.BlockSpec((BM, F), lambda i: (i, 0)),
            pl.BlockSpec((BM, F), lambda i: (i, 0)),
            pl.BlockSpec((1, F), lambda i: (0, 0)),
            pl.BlockSpec((F, F), lambda i: (0, 0)),
        ],
        out_specs=pl.BlockSpec((BM, F), lambda i: (i, 0)),
        out_shape=jax.ShapeDtypeStruct((NPAD, F), jnp.float32),
    )(P, g1, dinv_b, b1, W2)


def _tc3_body(p_ref, g2_ref, dinv_ref, b2_ref, wl1_ref, bl1_ref,
              wl2_ref, bl2_ref, y_ref):
    dinv = dinv_ref[...]
    t = jnp.tanh((p_ref[0] + p_ref[1] + g2_ref[...]) * dinv + b2_ref[...])
    m = jnp.maximum(
        jnp.dot(t, wl1_ref[...], preferred_element_type=jnp.float32)
        + bl1_ref[...], 0.0)
    y_ref[...] = jnp.dot(m, wl2_ref[...],
                         preferred_element_type=jnp.float32) + bl2_ref[...]


def _tc3(P, g2, dinv_b, b2, Wl1, bl1, wl2p, bl2p):
    return pl.pallas_call(
        _tc3_body,
        grid=(GRID,),
        in_specs=[
            pl.BlockSpec((NCORES, BM, F), lambda i: (0, i, 0)),
            pl.BlockSpec((BM, F), lambda i: (i, 0)),
            pl.BlockSpec((BM, F), lambda i: (i, 0)),
            pl.BlockSpec((1, F), lambda i: (0, 0)),
            pl.BlockSpec((F, F), lambda i: (0, 0)),
            pl.BlockSpec((1, F), lambda i: (0, 0)),
            pl.BlockSpec((F, F), lambda i: (0, 0)),
            pl.BlockSpec((1, F), lambda i: (0, 0)),
        ],
        out_specs=pl.BlockSpec((BM, F), lambda i: (i, 0)),
        out_shape=jax.ShapeDtypeStruct((NPAD, F), jnp.float32),
    )(P, g2, dinv_b, b2, Wl1, bl1, wl2p, bl2p)


def kernel(x, edge_index, W1, b1, W2, b2, Wl1, bl1, Wl2, bl2):
    xp = jnp.pad(x, ((0, NPAD - N), (0, 0)))
    ep = jnp.pad(edge_index, ((0, 0), (0, EPAD - E)), constant_values=N)
    idx_pad = jnp.full((NW, CIDX - CPROC, CHUNK), N, jnp.int32)
    src3 = jnp.concatenate(
        [ep[0].reshape(NW, CPROC, CHUNK), idx_pad], axis=1)
    dst3 = jnp.concatenate(
        [ep[1].reshape(NW, CPROC, CHUNK), idx_pad], axis=1)
    degP = _deg_kernel()(jnp.ones((CHUNK, DEGW), jnp.float32), dst3)
    g1, dinv_b = _tc1(xp, W1, degP)
    P1 = _agg_kernel()(g1, src3, dst3)
    g2 = _tc2(P1, g1, dinv_b, b1.reshape(1, F), W2)
    P2 = _agg_kernel()(g2, src3, dst3)
    wl2p = jnp.pad(Wl2, ((0, 0), (0, F - OUT)))
    bl2p = jnp.pad(bl2, (0, F - OUT)).reshape(1, F)
    y = _tc3(P2, g2, dinv_b, b2.reshape(1, F), Wl1,
             bl1.reshape(1, F), wl2p, bl2p)
    return y[:N, :OUT].reshape(-1, 1500, 2)


# speedup vs baseline: 1.1510x; 1.1510x over previous
import functools

import jax
import jax.numpy as jnp
from jax import lax
from jax.experimental import pallas as pl
from jax.experimental.pallas import tpu as pltpu
from jax.experimental.pallas import tpu_sc as plsc

N = 10000
E = 320000
F = 128
OUT = 60

NCORES = 2
NSUB = 16
NW = NCORES * NSUB
NPAD = 10240
ROWS_PER_TILE = NPAD // NSUB
CHUNK = 128
CPROC = 80
BLK = 8
CIDX = CPROC + 2 * BLK
EPAD = NW * CPROC * CHUNK
DEGW = 128

@functools.cache
def _deg_kernel():
  mesh = plsc.VectorSubcoreMesh(core_axis_name="c", subcore_axis_name="s",
                                num_cores=NCORES, num_subcores=NSUB)

  @functools.partial(
      pl.kernel,
      out_type=jax.ShapeDtypeStruct((NCORES, NPAD, DEGW), jnp.float32),
      mesh=mesh,
      scratch_types=[
          pltpu.VMEM((CIDX, CHUNK), jnp.int32),
          pltpu.VMEM((CHUNK, DEGW), jnp.float32),
          pltpu.VMEM((8, DEGW), jnp.float32),
          pltpu.VMEM_SHARED((NPAD, DEGW), jnp.float32),
      ],
  )
  def deg_kernel(ones_hbm, dst_hbm, out_hbm, dst_v, ones_v, zb_v, acc_sh):
    c = lax.axis_index("c")
    s = lax.axis_index("s")
    wid = c * NSUB + s
    for i in range(8):
        for j in range(DEGW // 16):
            zb_v[i, pl.ds(j * 16, 16)] = jnp.zeros((16,), jnp.float32)
    pltpu.sync_copy(ones_hbm, ones_v)
    row0 = s * ROWS_PER_TILE
    for k in range(ROWS_PER_TILE // 8):
        pltpu.sync_copy(zb_v, acc_sh.at[pl.ds(row0 + k * 8, 8), :])
    plsc.subcore_barrier()
    pltpu.sync_copy(dst_hbm.at[wid], dst_v)

    def body(ci, carry):
        pltpu.sync_copy(ones_v, acc_sh.at[dst_v.at[ci]], add=True)
        return carry

    lax.fori_loop(0, CPROC, body, 0)
    plsc.subcore_barrier()
    pltpu.sync_copy(acc_sh.at[pl.ds(row0, ROWS_PER_TILE), :],
                    out_hbm.at[c, pl.ds(row0, ROWS_PER_TILE), :])

  return deg_kernel


@functools.cache
def _agg_kernel():
  mesh = plsc.VectorSubcoreMesh(core_axis_name="c", subcore_axis_name="s",
                                num_cores=NCORES, num_subcores=NSUB)

  @functools.partial(
      pl.kernel,
      out_type=jax.ShapeDtypeStruct((NCORES, NPAD, F), jnp.float32),
      mesh=mesh,
      scratch_types=[
          pltpu.VMEM((CIDX, CHUNK), jnp.int32),
          pltpu.VMEM((CIDX, CHUNK), jnp.int32),
          pltpu.VMEM((CHUNK, F), jnp.float32),
          pltpu.VMEM((8, F), jnp.float32),
          pltpu.VMEM_SHARED((NPAD, F), jnp.float32),
          pltpu.SemaphoreType.DMA,
      ],
  )
  def agg_kernel(g_hbm, src_hbm, dst_hbm, out_hbm,
                 src_v, dst_v, rows_v, zb_v, acc_sh, sem):
    c = lax.axis_index("c")
    s = lax.axis_index("s")
    wid = c * NSUB + s
    for i in range(8):
        for j in range(F // 16):
            zb_v[i, pl.ds(j * 16, 16)] = jnp.zeros((16,), jnp.float32)
    row0 = s * ROWS_PER_TILE
    for k in range(ROWS_PER_TILE // 8):
        pltpu.sync_copy(zb_v, acc_sh.at[pl.ds(row0 + k * 8, 8), :])
    plsc.subcore_barrier()
    pltpu.sync_copy(src_hbm.at[wid], src_v)
    pltpu.sync_copy(dst_hbm.at[wid], dst_v)

    def body(ci, carry):
        pltpu.async_copy(g_hbm.at[src_v.at[ci]], rows_v, sem).wait()
        pltpu.sync_copy(rows_v, acc_sh.at[dst_v.at[ci]], add=True)
        return carry

    lax.fori_loop(0, CPROC, body, 0)
    plsc.subcore_barrier()
    pltpu.sync_copy(acc_sh.at[pl.ds(row0, ROWS_PER_TILE), :],
                    out_hbm.at[c, pl.ds(row0, ROWS_PER_TILE), :])

  return agg_kernel


BM = 1024
GRID = NPAD // BM


def _tc1_body(x_ref, w1_ref, deg_ref, g1_ref, dinv_ref):
    deg = deg_ref[0, :, :] + deg_ref[1, :, :]
    dinv = lax.rsqrt(deg[:, 0:1] + 1.0)
    h = jnp.dot(x_ref[...], w1_ref[...], preferred_element_type=jnp.float32)
    g1_ref[...] = h * dinv
    dinv_ref[...] = jnp.broadcast_to(dinv, (BM, F))


def _tc1(xp, W1, degP):
    return pl.pallas_call(
        _tc1_body,
        grid=(GRID,),
        in_specs=[
            pl.BlockSpec((BM, F), lambda i: (i, 0)),
            pl.BlockSpec((F, F), lambda i: (0, 0)),
            pl.BlockSpec((NCORES, BM, DEGW), lambda i: (0, i, 0)),
        ],
        out_specs=[
            pl.BlockSpec((BM, F), lambda i: (i, 0)),
            pl.BlockSpec((BM, F), lambda i: (i, 0)),
        ],
        out_shape=[
            jax.ShapeDtypeStruct((NPAD, F), jnp.float32),
            jax.ShapeDtypeStruct((NPAD, F), jnp.float32),
        ],
    )(xp, W1, degP)


def _tc2_body(p_ref, g1_ref, dinv_ref, b1_ref, w2_ref, g2_ref):
    dinv = dinv_ref[...]
    t = jnp.tanh((p_ref[0] + p_ref[1] + g1_ref[...]) * dinv + b1_ref[...])
    g2_ref[...] = jnp.dot(t, w2_ref[...],
                          preferred_element_type=jnp.float32) * dinv


def _tc2(P, g1, dinv_b, b1, W2):
    return pl.pallas_call(
        _tc2_body,
        grid=(GRID,),
        in_specs=[
            pl.BlockSpec((NCORES, BM, F), lambda i: (0, i, 0)),
            pl.BlockSpec((BM, F), lambda i: (i, 0)),
            pl.BlockSpec((BM, F), lambda i: (i, 0)),
            pl.BlockSpec((1, F), lambda i: (0, 0)),
            pl.BlockSpec((F, F), lambda i: (0, 0)),
        ],
        out_specs=pl.BlockSpec((BM, F), lambda i: (i, 0)),
        out_shape=jax.ShapeDtypeStruct((NPAD, F), jnp.float32),
    )(P, g1, dinv_b, b1, W2)


def _tc3_body(p_ref, g2_ref, dinv_ref, b2_ref, wl1_ref, bl1_ref,
              wl2_ref, bl2_ref, y_ref):
    dinv = dinv_ref[...]
    t = jnp.tanh((p_ref[0] + p_ref[1] + g2_ref[...]) * dinv + b2_ref[...])
    m = jnp.maximum(
        jnp.dot(t, wl1_ref[...], preferred_element_type=jnp.float32)
        + bl1_ref[...], 0.0)
    y_ref[...] = jnp.dot(m, wl2_ref[...],
                         preferred_element_type=jnp.float32) + bl2_ref[...]


def _tc3(P, g2, dinv_b, b2, Wl1, bl1, wl2p, bl2p):
    return pl.pallas_call(
        _tc3_body,
        grid=(GRID,),
        in_specs=[
            pl.BlockSpec((NCORES, BM, F), lambda i: (0, i, 0)),
            pl.BlockSpec((BM, F), lambda i: (i, 0)),
            pl.BlockSpec((BM, F), lambda i: (i, 0)),
            pl.BlockSpec((1, F), lambda i: (0, 0)),
            pl.BlockSpec((F, F), lambda i: (0, 0)),
            pl.BlockSpec((1, F), lambda i: (0, 0)),
            pl.BlockSpec((F, F), lambda i: (0, 0)),
            pl.BlockSpec((1, F), lambda i: (0, 0)),
        ],
        out_specs=pl.BlockSpec((BM, F), lambda i: (i, 0)),
        out_shape=jax.ShapeDtypeStruct((NPAD, F), jnp.float32),
    )(P, g2, dinv_b, b2, Wl1, bl1, wl2p, bl2p)


def kernel(x, edge_index, W1, b1, W2, b2, Wl1, bl1, Wl2, bl2):
    xp = jnp.pad(x, ((0, NPAD - N), (0, 0)))
    ep = jnp.pad(edge_index, ((0, 0), (0, EPAD - E)), constant_values=N)
    idx_pad = jnp.full((NW, CIDX - CPROC, CHUNK), N, jnp.int32)
    src3 = jnp.concatenate(
        [ep[0].reshape(NW, CPROC, CHUNK), idx_pad], axis=1)
    dst3 = jnp.concatenate(
        [ep[1].reshape(NW, CPROC, CHUNK), idx_pad], axis=1)
    degP = _deg_kernel()(jnp.ones((CHUNK, DEGW), jnp.float32), dst3)
    g1, dinv_b = _tc1(xp, W1, degP)
    P1 = _agg_kernel()(g1, src3, dst3)
    g2 = _tc2(P1, g1, dinv_b, b1.reshape(1, F), W2)
    P2 = _agg_kernel()(g2, src3, dst3)
    wl2p = jnp.pad(Wl2, ((0, 0), (0, F - OUT)))
    bl2p = jnp.pad(bl2, (0, F - OUT)).reshape(1, F)
    y = _tc3(P2, g2, dinv_b, b2.reshape(1, F), Wl1,
             bl1.reshape(1, F), wl2p, bl2p)
    return y[:N, :OUT].reshape(-1, 1500, 2)


# speedup vs baseline: 2.5641x; 2.2278x over previous
import functools

import jax
import jax.numpy as jnp
from jax import lax
from jax.experimental import pallas as pl
from jax.experimental.pallas import tpu as pltpu
from jax.experimental.pallas import tpu_sc as plsc

N = 10000
E = 320000
F = 128
OUT = 60

NCORES = 2
NSUB = 16
NW = NCORES * NSUB
NPAD = 10240
ROWS_PER_TILE = NPAD // NSUB
CHUNK = 128
CPROC = 80
BLK = 8
CIDX = CPROC + 2 * BLK
EPAD = NW * CPROC * CHUNK
DEGW = 128

@functools.cache
def _deg_kernel():
  mesh = plsc.VectorSubcoreMesh(core_axis_name="c", subcore_axis_name="s",
                                num_cores=NCORES, num_subcores=NSUB)

  @functools.partial(
      pl.kernel,
      out_type=jax.ShapeDtypeStruct((NCORES, NPAD, DEGW), jnp.float32),
      mesh=mesh,
      scratch_types=[
          pltpu.VMEM((CPROC, CHUNK), jnp.int32),
          pltpu.VMEM((CHUNK, DEGW), jnp.float32),
          pltpu.VMEM((8, DEGW), jnp.float32),
          pltpu.VMEM_SHARED((NPAD, DEGW), jnp.float32),
      ],
  )
  def deg_kernel(ones_hbm, dst_hbm, out_hbm, dst_v, ones_v, zb_v, acc_sh):
    c = lax.axis_index("c")
    s = lax.axis_index("s")
    wid = c * NSUB + s
    for i in range(8):
        for j in range(DEGW // 16):
            zb_v[i, pl.ds(j * 16, 16)] = jnp.zeros((16,), jnp.float32)
    pltpu.sync_copy(ones_hbm, ones_v)
    row0 = s * ROWS_PER_TILE
    for k in range(ROWS_PER_TILE // 8):
        pltpu.sync_copy(zb_v, acc_sh.at[pl.ds(row0 + k * 8, 8), :])
    plsc.subcore_barrier()
    pltpu.sync_copy(dst_hbm.at[wid], dst_v)

    def body(ci, carry):
        pltpu.sync_copy(ones_v, acc_sh.at[dst_v.at[ci]], add=True)
        return carry

    lax.fori_loop(0, CPROC, body, 0)
    plsc.subcore_barrier()
    pltpu.sync_copy(acc_sh.at[pl.ds(row0, ROWS_PER_TILE), :],
                    out_hbm.at[c, pl.ds(row0, ROWS_PER_TILE), :])

  return deg_kernel


@functools.cache
def _agg_kernel():
  mesh = plsc.VectorSubcoreMesh(core_axis_name="c", subcore_axis_name="s",
                                num_cores=NCORES, num_subcores=NSUB)

  @functools.partial(
      pl.kernel,
      out_type=jax.ShapeDtypeStruct((NCORES, NPAD, F), jnp.float32),
      mesh=mesh,
      scratch_types=[
          pltpu.VMEM((CPROC, CHUNK), jnp.int32),
          pltpu.VMEM((CPROC, CHUNK), jnp.int32),
          pltpu.VMEM((CHUNK, F), jnp.float32),
          pltpu.VMEM((8, F), jnp.float32),
          pltpu.VMEM_SHARED((NPAD, F), jnp.float32),
          pltpu.SemaphoreType.DMA,
      ],
  )
  def agg_kernel(g_hbm, src_hbm, dst_hbm, out_hbm,
                 src_v, dst_v, rows_v, zb_v, acc_sh, sem):
    c = lax.axis_index("c")
    s = lax.axis_index("s")
    wid = c * NSUB + s
    for i in range(8):
        for j in range(F // 16):
            zb_v[i, pl.ds(j * 16, 16)] = jnp.zeros((16,), jnp.float32)
    row0 = s * ROWS_PER_TILE
    for k in range(ROWS_PER_TILE // 8):
        pltpu.sync_copy(zb_v, acc_sh.at[pl.ds(row0 + k * 8, 8), :])
    plsc.subcore_barrier()
    pltpu.sync_copy(src_hbm.at[wid], src_v)
    pltpu.sync_copy(dst_hbm.at[wid], dst_v)

    def body(ci, carry):
        pltpu.async_copy(g_hbm.at[src_v.at[ci]], rows_v, sem).wait()
        pltpu.sync_copy(rows_v, acc_sh.at[dst_v.at[ci]], add=True)
        return carry

    lax.fori_loop(0, CPROC, body, 0)
    plsc.subcore_barrier()
    pltpu.sync_copy(acc_sh.at[pl.ds(row0, ROWS_PER_TILE), :],
                    out_hbm.at[c, pl.ds(row0, ROWS_PER_TILE), :])

  return agg_kernel


BM = 1024
GRID = NPAD // BM


def _tc1_body(x_ref, w1_ref, deg_ref, g1_ref, dinv_ref):
    deg = deg_ref[0, :, :] + deg_ref[1, :, :]
    dinv = lax.rsqrt(deg[:, 0:1] + 1.0)
    h = jnp.dot(x_ref[...], w1_ref[...], preferred_element_type=jnp.float32)
    g1_ref[...] = h * dinv
    dinv_ref[...] = jnp.broadcast_to(dinv, (BM, F))


def _tc1(xp, W1, degP):
    return pl.pallas_call(
        _tc1_body,
        grid=(GRID,),
        in_specs=[
            pl.BlockSpec((BM, F), lambda i: (i, 0)),
            pl.BlockSpec((F, F), lambda i: (0, 0)),
            pl.BlockSpec((NCORES, BM, DEGW), lambda i: (0, i, 0)),
        ],
        out_specs=[
            pl.BlockSpec((BM, F), lambda i: (i, 0)),
            pl.BlockSpec((BM, F), lambda i: (i, 0)),
        ],
        out_shape=[
            jax.ShapeDtypeStruct((NPAD, F), jnp.float32),
            jax.ShapeDtypeStruct((NPAD, F), jnp.float32),
        ],
    )(xp, W1, degP)


def _tc2_body(p_ref, g1_ref, dinv_ref, b1_ref, w2_ref, g2_ref):
    dinv = dinv_ref[...]
    t = jnp.tanh((p_ref[0] + p_ref[1] + g1_ref[...]) * dinv + b1_ref[...])
    g2_ref[...] = jnp.dot(t, w2_ref[...],
                          preferred_element_type=jnp.float32) * dinv


def _tc2(P, g1, dinv_b, b1, W2):
    return pl.pallas_call(
        _tc2_body,
        grid=(GRID,),
        in_specs=[
            pl.BlockSpec((NCORES, BM, F), lambda i: (0, i, 0)),
            pl.BlockSpec((BM, F), lambda i: (i, 0)),
            pl.BlockSpec((BM, F), lambda i: (i, 0)),
            pl.BlockSpec((1, F), lambda i: (0, 0)),
            pl.BlockSpec((F, F), lambda i: (0, 0)),
        ],
        out_specs=pl.BlockSpec((BM, F), lambda i: (i, 0)),
        out_shape=jax.ShapeDtypeStruct((NPAD, F), jnp.float32),
    )(P, g1, dinv_b, b1, W2)


def _tc3_body(p_ref, g2_ref, dinv_ref, b2_ref, wl1_ref, bl1_ref,
              wl2_ref, bl2_ref, y_ref):
    dinv = dinv_ref[...]
    t = jnp.tanh((p_ref[0] + p_ref[1] + g2_ref[...]) * dinv + b2_ref[...])
    m = jnp.maximum(
        jnp.dot(t, wl1_ref[...], preferred_element_type=jnp.float32)
        + bl1_ref[...], 0.0)
    y_ref[...] = jnp.dot(m, wl2_ref[...],
                         preferred_element_type=jnp.float32) + bl2_ref[...]


def _tc3(P, g2, dinv_b, b2, Wl1, bl1, wl2p, bl2p):
    return pl.pallas_call(
        _tc3_body,
        grid=(GRID,),
        in_specs=[
            pl.BlockSpec((NCORES, BM, F), lambda i: (0, i, 0)),
            pl.BlockSpec((BM, F), lambda i: (i, 0)),
            pl.BlockSpec((BM, F), lambda i: (i, 0)),
            pl.BlockSpec((1, F), lambda i: (0, 0)),
            pl.BlockSpec((F, F), lambda i: (0, 0)),
            pl.BlockSpec((1, F), lambda i: (0, 0)),
            pl.BlockSpec((F, F), lambda i: (0, 0)),
            pl.BlockSpec((1, F), lambda i: (0, 0)),
        ],
        out_specs=pl.BlockSpec((BM, F), lambda i: (i, 0)),
        out_shape=jax.ShapeDtypeStruct((NPAD, F), jnp.float32),
    )(P, g2, dinv_b, b2, Wl1, bl1, wl2p, bl2p)


def kernel(x, edge_index, W1, b1, W2, b2, Wl1, bl1, Wl2, bl2):
    xp = jnp.pad(x, ((0, NPAD - N), (0, 0)))
    pad_ids = N + (jnp.arange(EPAD - E, dtype=jnp.int32) % (NPAD - N))
    src3 = jnp.concatenate([edge_index[0], pad_ids]).reshape(
        NW, CPROC, CHUNK)
    dst3 = jnp.concatenate([edge_index[1], pad_ids]).reshape(
        NW, CPROC, CHUNK)
    degP = _deg_kernel()(jnp.ones((CHUNK, DEGW), jnp.float32), dst3)
    g1, dinv_b = _tc1(xp, W1, degP)
    P1 = _agg_kernel()(g1, src3, dst3)
    g2 = _tc2(P1, g1, dinv_b, b1.reshape(1, F), W2)
    P2 = _agg_kernel()(g2, src3, dst3)
    wl2p = jnp.pad(Wl2, ((0, 0), (0, F - OUT)))
    bl2p = jnp.pad(bl2, (0, F - OUT)).reshape(1, F)
    y = _tc3(P2, g2, dinv_b, b2.reshape(1, F), Wl1,
             bl1.reshape(1, F), wl2p, bl2p)
    return y[:N, :OUT].reshape(-1, 1500, 2)
